# baseline (speedup 1.0000x reference)
import functools

import jax
import jax.numpy as jnp
from jax import lax
from jax.experimental import pallas as pl
from jax.experimental.pallas import tpu as pltpu
from jax.experimental.pallas import tpu_sc as plsc


@functools.lru_cache(maxsize=None)
def _make_gather2(V, D, B):
  info = plsc.get_sparse_core_info()
  NC, NS = info.num_cores, info.num_subcores
  NW = NC * NS
  assert B % (8 * NW) == 0
  b_per_w = B // NW
  mesh = plsc.VectorSubcoreMesh(core_axis_name="c", subcore_axis_name="s")

  @functools.partial(
      pl.kernel,
      mesh=mesh,
      out_type=(
          jax.ShapeDtypeStruct((B, D), jnp.float32),
          jax.ShapeDtypeStruct((B, D), jnp.float32),
      ),
      scratch_types=[
          pltpu.VMEM((b_per_w,), jnp.int32),
          pltpu.VMEM((b_per_w,), jnp.int32),
          pltpu.VMEM((b_per_w, D), jnp.float32),
          pltpu.VMEM((b_per_w, D), jnp.float32),
          pltpu.SemaphoreType.DMA,
          pltpu.SemaphoreType.DMA,
      ],
      compiler_params=pltpu.CompilerParams(use_tc_tiling_on_sc=False),
  )
  def gather2(in_words, out_words, in_tab, out_tab, out_a, out_b,
              idx_a, idx_b, rows_a, rows_b, sem_a, sem_b):
    wid = lax.axis_index("s") * NC + lax.axis_index("c")
    base = wid * b_per_w
    pltpu.sync_copy(in_words.at[pl.ds(base, b_per_w)], idx_a)
    pltpu.sync_copy(out_words.at[pl.ds(base, b_per_w)], idx_b)
    cp_a = pltpu.async_copy(in_tab.at[idx_a], rows_a, sem_a)
    cp_b = pltpu.async_copy(out_tab.at[idx_b], rows_b, sem_b)
    cp_a.wait()
    pltpu.sync_copy(rows_a, out_a.at[pl.ds(base, b_per_w)])
    cp_b.wait()
    pltpu.sync_copy(rows_b, out_b.at[pl.ds(base, b_per_w)])

  return gather2


def kernel(input_words, output_words, in_embed, out_embed):
  V, D = in_embed.shape
  B = input_words.shape[0]
  fn = _make_gather2(V, D, B)
  return fn(input_words.astype(jnp.int32), output_words.astype(jnp.int32),
            in_embed, out_embed)


# speedup vs baseline: 1.0046x; 1.0046x over previous
import functools

import jax
import jax.numpy as jnp
from jax import lax
from jax.experimental import pallas as pl
from jax.experimental.pallas import tpu as pltpu
from jax.experimental.pallas import tpu_sc as plsc


@functools.lru_cache(maxsize=None)
def _make_gather(V, D, B):
  info = plsc.get_sparse_core_info()
  NC, NS = info.num_cores, info.num_subcores
  NW = NC * NS
  assert B % (8 * NW) == 0
  b_per_w = B // NW
  mesh = plsc.VectorSubcoreMesh(core_axis_name="c", subcore_axis_name="s")

  @functools.partial(
      pl.kernel,
      mesh=mesh,
      out_type=jax.ShapeDtypeStruct((B, D), jnp.float32),
      scratch_types=[
          pltpu.VMEM((b_per_w,), jnp.int32),
          pltpu.VMEM((b_per_w, D), jnp.float32),
          pltpu.SemaphoreType.DMA,
      ],
      compiler_params=pltpu.CompilerParams(use_tc_tiling_on_sc=False),
  )
  def gather1(words, tab, out, idx_v, rows_v, sem):
    wid = lax.axis_index("s") * NC + lax.axis_index("c")
    base = wid * b_per_w
    pltpu.sync_copy(words.at[pl.ds(base, b_per_w)], idx_v)
    pltpu.async_copy(tab.at[idx_v], rows_v, sem).wait()
    pltpu.sync_copy(rows_v, out.at[pl.ds(base, b_per_w)])

  return gather1


def kernel(input_words, output_words, in_embed, out_embed):
  V, D = in_embed.shape
  B = input_words.shape[0]
  fn = _make_gather(V, D, B)
  va = fn(input_words.astype(jnp.int32), in_embed)
  vb = fn(output_words.astype(jnp.int32), out_embed)
  return va, vb
